# baseline (speedup 1.0000x reference)
import functools

import jax
import jax.numpy as jnp
from jax import lax
from jax.experimental import pallas as pl
from jax.experimental.pallas import tpu as pltpu
from jax.experimental.pallas import tpu_sc as plsc

DIM = 64
LANES = 16
BLK = 128


def _make_kernel(B, L, vocab):
    info = plsc.get_sparse_core_info()
    num_workers = info.num_cores * info.num_subcores
    nB = B // BLK
    assert nB * BLK == B and nB % 4 == 0
    n_units = L * (nB // 4)
    per_w = n_units // num_workers
    assert per_w * num_workers == n_units
    m_per_l = nB // 4

    mesh = plsc.VectorSubcoreMesh(core_axis_name="c", subcore_axis_name="s")

    @functools.partial(
        pl.kernel,
        mesh=mesh,
        out_type=jax.ShapeDtypeStruct((L * DIM, B), jnp.float32),
        scratch_types=[
            pltpu.VMEM((8, BLK), jnp.int32),
            pltpu.VMEM((BLK, 2 * DIM), jnp.float32),
            pltpu.VMEM((BLK, 2 * DIM), jnp.float32),
            pltpu.VMEM((DIM, BLK), jnp.float32),
            pltpu.SemaphoreType.DMA,
        ],
        compiler_params=pltpu.CompilerParams(needs_layout_passes=False),
    )
    def k(idx_hbm, table_hbm, out_hbm, idx_v, rows0, rows1, out_v, sem):
        wid = lax.axis_index("s") * info.num_cores + lax.axis_index("c")
        u0 = wid * per_w
        lane = lax.iota(jnp.int32, LANES)
        row16 = [lane + g * LANES for g in range(BLK // LANES)]

        def unit_body(u, carry):
            l = u // m_per_l
            m = u % m_per_l
            pltpu.sync_copy(
                idx_hbm.at[l, pl.ds(pl.multiple_of(m * 8, 8), 8)], idx_v)
            for bq in range(4):
                c0 = pltpu.async_copy(table_hbm.at[idx_v.at[2 * bq]],
                                      rows0, sem)
                c1 = pltpu.async_copy(table_hbm.at[idx_v.at[2 * bq + 1]],
                                      rows1, sem)
                c0.wait()
                c1.wait()

                def d_body(d, c2):
                    dcol = jnp.zeros((LANES,), jnp.int32) + d
                    for g in range(BLK // LANES):
                        s = pl.ds(g * LANES, LANES)
                        a = plsc.load_gather(rows0, [row16[g], dcol])
                        b = plsc.load_gather(rows1, [row16[g], dcol])
                        out_v[d, s] = a + b
                    return c2

                lax.fori_loop(0, DIM, d_body, 0, unroll=2)
                row0 = pl.multiple_of(l * DIM, 8)
                col0 = pl.multiple_of((m * 4 + bq) * BLK, BLK)
                pltpu.sync_copy(out_v,
                                out_hbm.at[pl.ds(row0, DIM),
                                           pl.ds(col0, BLK)])
            return carry

        lax.fori_loop(u0, u0 + per_w, unit_body, 0)

    return k


def kernel(batch_pos_list, table):
    B, L, P = batch_pos_list.shape
    assert P == 2
    V, D = table.shape
    assert D == DIM
    idx3 = (batch_pos_list.transpose(1, 2, 0)
            .reshape(L, P, B // BLK, BLK)
            .transpose(0, 2, 1, 3)
            .reshape(L, P * B // BLK, BLK))
    table_p = jnp.pad(table, ((0, 0), (0, 2 * DIM - D)))
    k = _make_kernel(B, L, V)
    out2d = k(idx3, table_p)
    return out2d.reshape(L, DIM, B).transpose(2, 0, 1)


# speedup vs baseline: 1.1711x; 1.1711x over previous
import functools

import jax
import jax.numpy as jnp
from jax import lax
from jax.experimental import pallas as pl
from jax.experimental.pallas import tpu as pltpu
from jax.experimental.pallas import tpu_sc as plsc

DIM = 64
LANES = 16
BLK = 128


def _make_kernel(B, L, vocab):
    info = plsc.get_sparse_core_info()
    num_workers = info.num_cores * info.num_subcores
    nB = B // BLK
    assert nB * BLK == B and nB % 4 == 0
    n_blocks = L * nB
    per_w = n_blocks // num_workers
    assert per_w * num_workers == n_blocks
    assert (per_w * 2) % 8 == 0

    mesh = plsc.VectorSubcoreMesh(core_axis_name="c", subcore_axis_name="s")

    @functools.partial(
        pl.kernel,
        mesh=mesh,
        out_type=jax.ShapeDtypeStruct((L * DIM, B), jnp.float32),
        scratch_types=[
            pltpu.VMEM((2 * per_w, BLK), jnp.int32),
            pltpu.VMEM((4, BLK, 2 * DIM), jnp.float32),
            pltpu.VMEM((DIM, BLK), jnp.float32),
            pltpu.SemaphoreType.DMA((2,)),
        ],
        compiler_params=pltpu.CompilerParams(needs_layout_passes=False),
    )
    def k(idx_hbm, table_hbm, out_hbm, idx_v, rows, out_v, semg):
        wid = lax.axis_index("s") * info.num_cores + lax.axis_index("c")
        lane = lax.iota(jnp.int32, LANES)
        row16 = [lane + g * LANES for g in range(BLK // LANES)]

        pltpu.sync_copy(
            idx_hbm.at[pl.ds(pl.multiple_of(wid * 2 * per_w, 8), 2 * per_w)],
            idx_v)

        def fire(kb, par):
            pltpu.async_copy(table_hbm.at[idx_v.at[2 * kb]],
                             rows.at[2 * par], semg.at[par])
            pltpu.async_copy(table_hbm.at[idx_v.at[2 * kb + 1]],
                             rows.at[2 * par + 1], semg.at[par])

        def wait(kb, par):
            pltpu.make_async_copy(table_hbm.at[idx_v.at[2 * kb]],
                                  rows.at[2 * par], semg.at[par]).wait()
            pltpu.make_async_copy(table_hbm.at[idx_v.at[2 * kb + 1]],
                                  rows.at[2 * par + 1], semg.at[par]).wait()

        def out_slice(kb):
            gb = wid * per_w + kb
            l = gb // nB
            c = gb % nB
            return out_hbm.at[
                pl.ds(pl.multiple_of(l * DIM, 8), DIM),
                pl.ds(pl.multiple_of(c * BLK, BLK), BLK)]

        fire(0, 0)

        def body(kb, carry):
            par = lax.rem(kb, 2)
            nxt = 1 - par

            @pl.when(kb < per_w - 1)
            def _():
                fire(kb + 1, nxt)

            wait(kb, par)
            r0 = rows.at[2 * par]
            r1 = rows.at[2 * par + 1]
            for d in range(DIM):
                dcol = jnp.zeros((LANES,), jnp.int32) + d
                for g in range(BLK // LANES):
                    a = plsc.load_gather(r0, [row16[g], dcol])
                    b = plsc.load_gather(r1, [row16[g], dcol])
                    out_v[d, pl.ds(g * LANES, LANES)] = a + b
            pltpu.sync_copy(out_v, out_slice(kb))
            return carry

        lax.fori_loop(0, per_w, body, 0)

    return k


def kernel(batch_pos_list, table):
    B, L, P = batch_pos_list.shape
    assert P == 2
    V, D = table.shape
    assert D == DIM
    idx2 = (batch_pos_list.transpose(1, 2, 0)
            .reshape(L, P, B // BLK, BLK)
            .transpose(0, 2, 1, 3)
            .reshape(L * P * (B // BLK), BLK))
    table_p = jnp.pad(table, ((0, 0), (0, 2 * DIM - D)))
    k = _make_kernel(B, L, V)
    out2d = k(idx2, table_p)
    return out2d.reshape(L, DIM, B).transpose(2, 0, 1)


# speedup vs baseline: 2.3766x; 2.0293x over previous
import functools

import jax
import jax.numpy as jnp
from jax import lax
from jax.experimental import pallas as pl
from jax.experimental.pallas import tpu as pltpu
from jax.experimental.pallas import tpu_sc as plsc

DIM = 64
LANES = 16
BLK = 128


def _make_kernel(B, L, vocab):
    info = plsc.get_sparse_core_info()
    num_workers = info.num_cores * info.num_subcores
    nB = B // BLK
    assert nB * BLK == B and nB % 4 == 0
    n_blocks = L * nB
    per_w = n_blocks // num_workers
    assert per_w * num_workers == n_blocks
    assert (per_w * 2) % 8 == 0

    mesh = plsc.VectorSubcoreMesh(core_axis_name="c", subcore_axis_name="s")

    @functools.partial(
        pl.kernel,
        mesh=mesh,
        out_type=jax.ShapeDtypeStruct((L * DIM, B), jnp.float32),
        scratch_types=[
            pltpu.VMEM((2 * per_w, BLK), jnp.int32),
            pltpu.VMEM((4, BLK, 2 * DIM), jnp.float32),
            pltpu.VMEM((DIM, BLK), jnp.float32),
            pltpu.SemaphoreType.DMA((2,)),
        ],
        compiler_params=pltpu.CompilerParams(needs_layout_passes=False),
    )
    def k(idx_hbm, table_hbm, out_hbm, idx_v, rows, out_v, semg):
        wid = lax.axis_index("s") * info.num_cores + lax.axis_index("c")
        lane = lax.iota(jnp.int32, LANES)
        row16 = [lane + g * LANES for g in range(BLK // LANES)]

        pltpu.sync_copy(
            idx_hbm.at[pl.ds(pl.multiple_of(wid * 2 * per_w, 8), 2 * per_w)],
            idx_v)

        def fire(kb, par):
            pltpu.async_copy(table_hbm.at[idx_v.at[2 * kb]],
                             rows.at[2 * par], semg.at[par])
            pltpu.async_copy(table_hbm.at[idx_v.at[2 * kb + 1]],
                             rows.at[2 * par + 1], semg.at[par])

        def wait(kb, par):
            pltpu.make_async_copy(table_hbm.at[idx_v.at[2 * kb]],
                                  rows.at[2 * par], semg.at[par]).wait()
            pltpu.make_async_copy(table_hbm.at[idx_v.at[2 * kb + 1]],
                                  rows.at[2 * par + 1], semg.at[par]).wait()

        def out_slice(kb):
            gb = wid * per_w + kb
            l = gb // nB
            c = gb % nB
            return out_hbm.at[
                pl.ds(pl.multiple_of(l * DIM, 8), DIM),
                pl.ds(pl.multiple_of(c * BLK, BLK), BLK)]

        fire(0, 0)

        def body(kb, carry):
            par = lax.rem(kb, 2)
            nxt = 1 - par

            @pl.when(kb < per_w - 1)
            def _():
                fire(kb + 1, nxt)

            wait(kb, par)
            r0 = rows.at[2 * par]
            r1 = rows.at[2 * par + 1]
            TG = 4
            nk = DIM // LANES
            for t0 in range(0, BLK, TG):
                loads0 = [r0[t0 + i, pl.ds(kk * LANES, LANES)]
                          for i in range(TG) for kk in range(nk)]
                loads1 = [r1[t0 + i, pl.ds(kk * LANES, LANES)]
                          for i in range(TG) for kk in range(nk)]
                tcols = [jnp.zeros((LANES,), jnp.int32) + (t0 + i)
                         for i in range(TG)]
                sums = [a + b for a, b in zip(loads0, loads1)]
                for i in range(TG):
                    for kk in range(nk):
                        plsc.store_scatter(out_v, [row16[kk], tcols[i]],
                                           sums[i * nk + kk])
            pltpu.sync_copy(out_v, out_slice(kb))
            return carry

        lax.fori_loop(0, per_w, body, 0)

    return k


def kernel(batch_pos_list, table):
    B, L, P = batch_pos_list.shape
    assert P == 2
    V, D = table.shape
    assert D == DIM
    idx2 = (batch_pos_list.transpose(1, 2, 0)
            .reshape(L, P, B // BLK, BLK)
            .transpose(0, 2, 1, 3)
            .reshape(L * P * (B // BLK), BLK))
    table_p = jnp.pad(table, ((0, 0), (0, 2 * DIM - D)))
    k = _make_kernel(B, L, V)
    out2d = k(idx2, table_p)
    return out2d.reshape(L, DIM, B).transpose(2, 0, 1)


# speedup vs baseline: 3.8313x; 1.6121x over previous
import functools

import jax
import jax.numpy as jnp
from jax import lax
from jax.experimental import pallas as pl
from jax.experimental.pallas import tpu as pltpu
from jax.experimental.pallas import tpu_sc as plsc

DIM = 64
LANES = 16
BLK = 128


def _make_kernel(B, L, vocab):
    info = plsc.get_sparse_core_info()
    num_workers = info.num_cores * info.num_subcores
    nB = B // BLK
    assert nB * BLK == B and nB % 4 == 0
    n_blocks = L * nB
    per_w = n_blocks // num_workers
    assert per_w * num_workers == n_blocks
    assert (per_w * 2) % 8 == 0

    mesh = plsc.VectorSubcoreMesh(core_axis_name="c", subcore_axis_name="s")

    @functools.partial(
        pl.kernel,
        mesh=mesh,
        out_type=jax.ShapeDtypeStruct((L * DIM, B), jnp.float32),
        scratch_types=[
            pltpu.VMEM((per_w, BLK), jnp.int32),
            pltpu.VMEM((4, BLK, 2 * DIM), jnp.float32),
            pltpu.VMEM((DIM, BLK), jnp.float32),
            pltpu.SemaphoreType.DMA((2,)),
        ],
        compiler_params=pltpu.CompilerParams(needs_layout_passes=False),
    )
    def k(idx_hbm, table_hbm, out_hbm, idx_v, rows, out_v, semg):
        wid = lax.axis_index("s") * info.num_cores + lax.axis_index("c")
        lane = lax.iota(jnp.int32, LANES)
        row16 = [lane + g * LANES for g in range(BLK // LANES)]

        HB = per_w // 2

        def load_idx_half(h):
            pltpu.sync_copy(
                idx_hbm.at[pl.ds(
                    pl.multiple_of(wid * 2 * per_w + h * 2 * HB, 8), 2 * HB)],
                idx_v)

        load_idx_half(0)

        def fire(kb, par):
            r = lax.rem(2 * kb, 2 * HB)
            pltpu.async_copy(table_hbm.at[idx_v.at[r]],
                             rows.at[2 * par], semg.at[par])
            pltpu.async_copy(table_hbm.at[idx_v.at[r + 1]],
                             rows.at[2 * par + 1], semg.at[par])

        def wait(kb, par):
            r = lax.rem(2 * kb, 2 * HB)
            pltpu.make_async_copy(table_hbm.at[idx_v.at[r]],
                                  rows.at[2 * par], semg.at[par]).wait()
            pltpu.make_async_copy(table_hbm.at[idx_v.at[r + 1]],
                                  rows.at[2 * par + 1], semg.at[par]).wait()

        def out_slice(kb):
            gb = wid * per_w + kb
            l = gb // nB
            c = gb % nB
            return out_hbm.at[
                pl.ds(pl.multiple_of(l * DIM, 8), DIM),
                pl.ds(pl.multiple_of(c * BLK, BLK), BLK)]

        fire(0, 0)

        def body(kb, carry):
            par = lax.rem(kb, 2)
            nxt = 1 - par

            @pl.when(jnp.logical_and(kb < per_w - 1, kb != HB - 1))
            def _():
                fire(kb + 1, nxt)

            wait(kb, par)

            @pl.when(kb == HB - 1)
            def _():
                load_idx_half(1)
                fire(kb + 1, nxt)
            r0 = rows.at[2 * par]
            r1 = rows.at[2 * par + 1]
            tvs = [lane + t0 for t0 in range(0, BLK, LANES)]

            def diag_body(j, c2):
                dv = ((lane + j) & (LANES - 1)) + (j & (DIM - LANES))
                for tv in tvs:
                    a = plsc.load_gather(r0, [tv, dv])
                    b = plsc.load_gather(r1, [tv, dv])
                    plsc.store_scatter(out_v, [dv, tv], a + b)
                return c2

            lax.fori_loop(0, DIM, diag_body, 0)
            pltpu.sync_copy(out_v, out_slice(kb))
            return carry

        lax.fori_loop(0, per_w, body, 0)

    return k


def kernel(batch_pos_list, table):
    B, L, P = batch_pos_list.shape
    assert P == 2
    V, D = table.shape
    assert D == DIM
    idx2 = (batch_pos_list.transpose(1, 2, 0)
            .reshape(L, P, B // BLK, BLK)
            .transpose(0, 2, 1, 3)
            .reshape(L * P * (B // BLK), BLK))
    table_p = jnp.pad(table, ((0, 0), (0, 2 * DIM - D)))
    k = _make_kernel(B, L, V)
    out2d = k(idx2, table_p)
    return out2d.reshape(L, DIM, B).transpose(2, 0, 1)


# speedup vs baseline: 3.8876x; 1.0147x over previous
import functools

import jax
import jax.numpy as jnp
from jax import lax
from jax.experimental import pallas as pl
from jax.experimental.pallas import tpu as pltpu
from jax.experimental.pallas import tpu_sc as plsc

DIM = 64
LANES = 16
BLK = 128


def _make_kernel(B, L, vocab):
    info = plsc.get_sparse_core_info()
    num_workers = info.num_cores * info.num_subcores
    nB = B // BLK
    assert nB * BLK == B and nB % 4 == 0
    n_blocks = L * nB
    per_w = n_blocks // num_workers
    assert per_w * num_workers == n_blocks
    assert (per_w * 2) % 8 == 0

    mesh = plsc.VectorSubcoreMesh(core_axis_name="c", subcore_axis_name="s")

    @functools.partial(
        pl.kernel,
        mesh=mesh,
        out_type=jax.ShapeDtypeStruct((L * DIM, B), jnp.float32),
        scratch_types=[
            pltpu.VMEM((per_w, BLK), jnp.int32),
            pltpu.VMEM((4, BLK, 2 * DIM), jnp.float32),
            pltpu.VMEM((DIM, BLK), jnp.float32),
            pltpu.SemaphoreType.DMA((2,)),
        ],
        compiler_params=pltpu.CompilerParams(needs_layout_passes=False),
    )
    def k(idx_hbm, table_hbm, out_hbm, idx_v, rows, out_v, semg):
        wid = lax.axis_index("s") * info.num_cores + lax.axis_index("c")
        lane = lax.iota(jnp.int32, LANES)
        row16 = [lane + g * LANES for g in range(BLK // LANES)]

        HB = per_w // 2

        def load_idx_half(h):
            pltpu.sync_copy(
                idx_hbm.at[pl.ds(
                    pl.multiple_of(wid * 2 * per_w + h * 2 * HB, 8), 2 * HB)],
                idx_v)

        load_idx_half(0)

        def fire(kb, par):
            r = lax.rem(2 * kb, 2 * HB)
            pltpu.async_copy(table_hbm.at[idx_v.at[r]],
                             rows.at[2 * par], semg.at[par])
            pltpu.async_copy(table_hbm.at[idx_v.at[r + 1]],
                             rows.at[2 * par + 1], semg.at[par])

        def wait(kb, par):
            r = lax.rem(2 * kb, 2 * HB)
            pltpu.make_async_copy(table_hbm.at[idx_v.at[r]],
                                  rows.at[2 * par], semg.at[par]).wait()
            pltpu.make_async_copy(table_hbm.at[idx_v.at[r + 1]],
                                  rows.at[2 * par + 1], semg.at[par]).wait()

        def out_slice(kb):
            gb = wid * per_w + kb
            l = gb // nB
            c = gb % nB
            return out_hbm.at[
                pl.ds(pl.multiple_of(l * DIM, 8), DIM),
                pl.ds(pl.multiple_of(c * BLK, BLK), BLK)]

        fire(0, 0)

        def body(kb, carry):
            par = lax.rem(kb, 2)
            nxt = 1 - par

            @pl.when(jnp.logical_and(kb < per_w - 1, kb != HB - 1))
            def _():
                fire(kb + 1, nxt)

            wait(kb, par)

            @pl.when(kb == HB - 1)
            def _():
                load_idx_half(1)
                fire(kb + 1, nxt)
            r0 = rows.at[2 * par]
            r1 = rows.at[2 * par + 1]
            tvs = [lane + t0 for t0 in range(0, BLK, LANES)]

            def diag_body(j, c2):
                dv = ((lane + j) & (LANES - 1)) + (j & (DIM - LANES))
                for tv in tvs:
                    a = plsc.load_gather(r0, [tv, dv])
                    b = plsc.load_gather(r1, [tv, dv])
                    plsc.store_scatter(out_v, [dv, tv], a + b)
                return c2

            lax.fori_loop(0, DIM, diag_body, 0, unroll=2)
            pltpu.sync_copy(out_v, out_slice(kb))
            return carry

        lax.fori_loop(0, per_w, body, 0)

    return k


def kernel(batch_pos_list, table):
    B, L, P = batch_pos_list.shape
    assert P == 2
    V, D = table.shape
    assert D == DIM
    idx2 = (batch_pos_list.transpose(1, 2, 0)
            .reshape(L, P, B // BLK, BLK)
            .transpose(0, 2, 1, 3)
            .reshape(L * P * (B // BLK), BLK))
    table_p = jnp.pad(table, ((0, 0), (0, 2 * DIM - D)))
    k = _make_kernel(B, L, V)
    out2d = k(idx2, table_p)
    return out2d.reshape(L, DIM, B).transpose(2, 0, 1)
